# Initial kernel scaffold; baseline (speedup 1.0000x reference)
import jax
import jax.numpy as jnp
from jax.experimental import pallas as pl


def kernel(x0, x1, x2, x3, W0, W1, W2, W3):
    raise NotImplementedError("write your pallas kernel here")


# speedup vs baseline: 5.9911x; 5.9911x over previous
import functools

import jax
import jax.numpy as jnp
from jax import lax
from jax.experimental import pallas as pl
from jax.experimental.pallas import tpu as pltpu
from jax.experimental.pallas import tpu_sc as plsc

_B, _T = 4096, 50
_NTOT = _B * _T
_DIMS = (64, 64, 32, 32)
_OFFS = (0, 64, 128, 160)
_DSUM = 192
_NC, _NS = 2, 16
_NW = _NC * _NS
_BPW = _NTOT // _NW
_CHUNK = 128
_NCH = _BPW // _CHUNK

_mesh = plsc.VectorSubcoreMesh(core_axis_name="c", subcore_axis_name="s")


@functools.partial(
    pl.kernel,
    out_type=jax.ShapeDtypeStruct((_NTOT, _DSUM), jnp.float32),
    mesh=_mesh,
    compiler_params=pltpu.CompilerParams(use_tc_tiling_on_sc=False),
    scratch_types=[
        pltpu.VMEM((_BPW,), jnp.int32),
        pltpu.VMEM((_BPW,), jnp.int32),
        pltpu.VMEM((_BPW,), jnp.int32),
        pltpu.VMEM((_BPW,), jnp.int32),
        pltpu.VMEM((_CHUNK, 64), jnp.float32),
        pltpu.VMEM((_CHUNK, 64), jnp.float32),
        pltpu.VMEM((_CHUNK, 32), jnp.float32),
        pltpu.VMEM((_CHUNK, 32), jnp.float32),
        pltpu.SemaphoreType.DMA,
        pltpu.SemaphoreType.DMA,
    ],
)
def _emb_gather(x0_h, x1_h, x2_h, x3_h, w0_h, w1_h, w2_h, w3_h, out_h,
                i0, i1, i2, i3, s0, s1, s2, s3, gsem, osem):
    wid = lax.axis_index("s") * _NC + lax.axis_index("c")
    base = wid * _BPW

    pltpu.sync_copy(x0_h.at[pl.ds(base, _BPW)], i0)
    pltpu.sync_copy(x1_h.at[pl.ds(base, _BPW)], i1)
    pltpu.sync_copy(x2_h.at[pl.ds(base, _BPW)], i2)
    pltpu.sync_copy(x3_h.at[pl.ds(base, _BPW)], i3)

    idx_refs = (i0, i1, i2, i3)
    w_refs = (w0_h, w1_h, w2_h, w3_h)
    stages = (s0, s1, s2, s3)

    def chunk_body(j):
        off = pl.multiple_of(j * _CHUNK, _CHUNK)
        handles = []
        for k in range(4):
            src = w_refs[k].at[idx_refs[k].at[pl.ds(off, _CHUNK)]]
            handles.append(pltpu.async_copy(src, stages[k], gsem))
        for h in handles:
            h.wait()
        ohandles = []
        for k in range(4):
            dst = out_h.at[pl.ds(base + off, _CHUNK), pl.ds(_OFFS[k], _DIMS[k])]
            ohandles.append(pltpu.async_copy(stages[k], dst, osem))
        for h in ohandles:
            h.wait()

    pl.loop(0, _NCH)(chunk_body)


def kernel(x0, x1, x2, x3, W0, W1, W2, W3):
    xs = [x.reshape(-1).astype(jnp.int32) for x in (x0, x1, x2, x3)]
    out = _emb_gather(xs[0], xs[1], xs[2], xs[3], W0, W1, W2, W3)
    return out.reshape(_B, _T, _DSUM)
